# Initial kernel scaffold; baseline (speedup 1.0000x reference)
import jax
import jax.numpy as jnp
from jax.experimental import pallas as pl


def kernel(x, edge_index, batch, emb_W, emb_b, conv1_W, conv1_att_src, conv1_att_dst, conv1_b, conv2_W, conv2_att_src, conv2_att_dst, conv2_b, fc_W, fc_b, out_W, out_b):
    raise NotImplementedError("write your pallas kernel here")


# speedup vs baseline: 46.2820x; 46.2820x over previous
import functools

import jax
import jax.numpy as jnp
from jax import lax
from jax.experimental import pallas as pl
from jax.experimental.pallas import tpu as pltpu
from jax.experimental.pallas import tpu_sc as plsc

N = 100000
E = 1600000
D_IN = 128
D_HID = 32
HALF = D_HID // 2
NG = 64

NSC = 2
NTILE = 16

NPAD = 100352
RPT = NPAD // NTILE

CW = 128
BLK = 8
CHUNKS = 12544
EPAD = CHUNKS * CW
CPT = CHUNKS // NTILE
NBLK = CPT // BLK

BN = 1024
GRID = NPAD // BN


def _leaky(t, slope):
    return jnp.where(t < 0, t * slope, t)


def _embed_body(x_ref, embw_ref, embb_ref, w1_ref, att_ref,
                zlo_ref, zhi_ref, as_ref, ad_ref):
    h = jnp.dot(x_ref[...], embw_ref[...],
                preferred_element_type=jnp.float32) + embb_ref[...]
    z = jnp.dot(h, w1_ref[...], preferred_element_type=jnp.float32)
    a = jnp.dot(z, att_ref[...], preferred_element_type=jnp.float32)
    zlo_ref[...] = z[:, :HALF]
    zhi_ref[...] = z[:, HALF:]
    as_ref[...] = a[:, 0:1]
    ad_ref[...] = a[:, 1:2]


_embed = pl.pallas_call(
    _embed_body,
    grid=(GRID,),
    in_specs=[
        pl.BlockSpec((BN, D_IN), lambda i: (i, 0)),
        pl.BlockSpec((D_IN, D_HID), lambda i: (0, 0)),
        pl.BlockSpec((1, D_HID), lambda i: (0, 0)),
        pl.BlockSpec((D_HID, D_HID), lambda i: (0, 0)),
        pl.BlockSpec((D_HID, 2), lambda i: (0, 0)),
    ],
    out_specs=[
        pl.BlockSpec((BN, HALF), lambda i: (i, 0)),
        pl.BlockSpec((BN, HALF), lambda i: (i, 0)),
        pl.BlockSpec((BN, 1), lambda i: (i, 0)),
        pl.BlockSpec((BN, 1), lambda i: (i, 0)),
    ],
    out_shape=[
        jax.ShapeDtypeStruct((NPAD, HALF), jnp.float32),
        jax.ShapeDtypeStruct((NPAD, HALF), jnp.float32),
        jax.ShapeDtypeStruct((NPAD, 1), jnp.float32),
        jax.ShapeDtypeStruct((NPAD, 1), jnp.float32),
    ],
)


def _mid_body(nlo_ref, nhi_ref, den_ref, as_ref, ad_ref, zlo_ref, zhi_ref,
              b1_ref, w2_ref, att2_ref,
              zlo2_ref, zhi2_ref, as2_ref, ad2_ref):
    t = as_ref[...] + ad_ref[...]
    wself = jnp.exp(_leaky(t, 0.2))
    z1 = jnp.concatenate([zlo_ref[...], zhi_ref[...]], axis=1)
    num = jnp.concatenate([nlo_ref[...], nhi_ref[...]], axis=1)
    h1 = (num + wself * z1) / (den_ref[...] + wself) + b1_ref[...]
    z2 = jnp.dot(h1, w2_ref[...], preferred_element_type=jnp.float32)
    a2 = jnp.dot(z2, att2_ref[...], preferred_element_type=jnp.float32)
    zlo2_ref[...] = z2[:, :HALF]
    zhi2_ref[...] = z2[:, HALF:]
    as2_ref[...] = a2[:, 0:1]
    ad2_ref[...] = a2[:, 1:2]


_mid = pl.pallas_call(
    _mid_body,
    grid=(GRID,),
    in_specs=[
        pl.BlockSpec((BN, HALF), lambda i: (i, 0)),
        pl.BlockSpec((BN, HALF), lambda i: (i, 0)),
        pl.BlockSpec((BN, 1), lambda i: (i, 0)),
        pl.BlockSpec((BN, 1), lambda i: (i, 0)),
        pl.BlockSpec((BN, 1), lambda i: (i, 0)),
        pl.BlockSpec((BN, HALF), lambda i: (i, 0)),
        pl.BlockSpec((BN, HALF), lambda i: (i, 0)),
        pl.BlockSpec((1, D_HID), lambda i: (0, 0)),
        pl.BlockSpec((D_HID, D_HID), lambda i: (0, 0)),
        pl.BlockSpec((D_HID, 2), lambda i: (0, 0)),
    ],
    out_specs=[
        pl.BlockSpec((BN, HALF), lambda i: (i, 0)),
        pl.BlockSpec((BN, HALF), lambda i: (i, 0)),
        pl.BlockSpec((BN, 1), lambda i: (i, 0)),
        pl.BlockSpec((BN, 1), lambda i: (i, 0)),
    ],
    out_shape=[
        jax.ShapeDtypeStruct((NPAD, HALF), jnp.float32),
        jax.ShapeDtypeStruct((NPAD, HALF), jnp.float32),
        jax.ShapeDtypeStruct((NPAD, 1), jnp.float32),
        jax.ShapeDtypeStruct((NPAD, 1), jnp.float32),
    ],
)


def _final_body(nlo_ref, nhi_ref, den_ref, as_ref, ad_ref, zlo_ref, zhi_ref,
                b2_ref, batch_ref, fcw_ref, fcb_ref, outw_ref, outb_ref,
                proba_ref, logits_ref, sums_ref, cnt_ref):
    i = pl.program_id(0)
    t = as_ref[...] + ad_ref[...]
    wself = jnp.exp(_leaky(t, 0.2))
    z2 = jnp.concatenate([zlo_ref[...], zhi_ref[...]], axis=1)
    num = jnp.concatenate([nlo_ref[...], nhi_ref[...]], axis=1)
    h2 = (num + wself * z2) / (den_ref[...] + wself) + b2_ref[...]
    b = batch_ref[0]
    gid = lax.broadcasted_iota(jnp.int32, (NG, BN), 0)
    oh = (gid == jnp.broadcast_to(b, (NG, BN))).astype(jnp.float32)
    psum = jnp.dot(oh, h2, preferred_element_type=jnp.float32)
    pcnt = jnp.sum(oh, axis=1, keepdims=True)

    @pl.when(i == 0)
    def _():
        sums_ref[...] = psum
        cnt_ref[...] = pcnt

    @pl.when(i > 0)
    def _():
        sums_ref[...] += psum
        cnt_ref[...] += pcnt

    @pl.when(i == GRID - 1)
    def _():
        gf = sums_ref[...] / jnp.maximum(cnt_ref[...], 1.0)
        g = jnp.dot(gf, fcw_ref[...],
                    preferred_element_type=jnp.float32) + fcb_ref[...]
        g = _leaky(g, 0.01)
        o = jnp.dot(g, outw_ref[...],
                    preferred_element_type=jnp.float32) + outb_ref[...]
        m = jnp.max(o, axis=-1, keepdims=True)
        e = jnp.exp(o - m)
        proba_ref[...] = e / jnp.sum(e, axis=-1, keepdims=True)
        logits_ref[...] = o


_final = pl.pallas_call(
    _final_body,
    grid=(GRID,),
    in_specs=[
        pl.BlockSpec((BN, HALF), lambda i: (i, 0)),
        pl.BlockSpec((BN, HALF), lambda i: (i, 0)),
        pl.BlockSpec((BN, 1), lambda i: (i, 0)),
        pl.BlockSpec((BN, 1), lambda i: (i, 0)),
        pl.BlockSpec((BN, 1), lambda i: (i, 0)),
        pl.BlockSpec((BN, HALF), lambda i: (i, 0)),
        pl.BlockSpec((BN, HALF), lambda i: (i, 0)),
        pl.BlockSpec((1, D_HID), lambda i: (0, 0)),
        pl.BlockSpec((1, 1, BN), lambda i: (i, 0, 0)),
        pl.BlockSpec((D_HID, 64), lambda i: (0, 0)),
        pl.BlockSpec((1, 64), lambda i: (0, 0)),
        pl.BlockSpec((64, 2), lambda i: (0, 0)),
        pl.BlockSpec((1, 2), lambda i: (0, 0)),
    ],
    out_specs=[
        pl.BlockSpec((NG, 2), lambda i: (0, 0)),
        pl.BlockSpec((NG, 2), lambda i: (0, 0)),
    ],
    out_shape=[
        jax.ShapeDtypeStruct((NG, 2), jnp.float32),
        jax.ShapeDtypeStruct((NG, 2), jnp.float32),
    ],
    scratch_shapes=[
        pltpu.VMEM((NG, D_HID), jnp.float32),
        pltpu.VMEM((NG, 1), jnp.float32),
    ],
)


def _make_edge_kernel():
    mesh = plsc.VectorSubcoreMesh(
        core_axis_name="c", subcore_axis_name="s",
        num_cores=NSC, num_subcores=NTILE)

    @functools.partial(
        pl.kernel,
        out_type=[
            jax.ShapeDtypeStruct((NPAD, HALF), jnp.float32),
            jax.ShapeDtypeStruct((NPAD, HALF), jnp.float32),
            jax.ShapeDtypeStruct((NPAD,), jnp.float32),
        ],
        mesh=mesh,
        compiler_params=pltpu.CompilerParams(use_tc_tiling_on_sc=False),
        scratch_types=[
            pltpu.VMEM((BLK, CW), jnp.int32),
            pltpu.VMEM((BLK, CW), jnp.int32),
            pltpu.VMEM((BLK, CW), jnp.float32),
            pltpu.VMEM((BLK, CW), jnp.float32),
            pltpu.VMEM((BLK, CW), jnp.float32),
            pltpu.VMEM((BLK, CW, HALF), jnp.float32),
            pltpu.VMEM_SHARED((NPAD, HALF), jnp.float32),
            pltpu.VMEM_SHARED((NPAD,), jnp.float32),
            pltpu.SemaphoreType.DMA,
            pltpu.SemaphoreType.DMA,
        ],
    )
    def edge_kernel(zlo, zhi, asrc, adst, ei, zrow, zcol,
                    nlo, nhi, dout,
                    srcb, dstb, asb, adb, wb, zb, acc, dacc, gsem, ssem):
        c = lax.axis_index("c")
        s = lax.axis_index("s")

        pltpu.sync_copy(zrow, acc.at[pl.ds(s * RPT, RPT)])

        @pl.when(c == 0)
        def _():
            pltpu.sync_copy(zcol, dacc.at[pl.ds(s * RPT, RPT)])

        plsc.subcore_barrier()

        def run(ztab, with_den):
            def block_body(bi, carry):
                ch0 = s * CPT + bi * BLK
                pltpu.sync_copy(ei.at[0, pl.ds(ch0, BLK)], srcb)
                pltpu.sync_copy(ei.at[1, pl.ds(ch0, BLK)], dstb)
                gathers = []
                for j in range(BLK):
                    gathers.append(
                        pltpu.async_copy(asrc.at[srcb.at[j]], asb.at[j], gsem))
                    gathers.append(
                        pltpu.async_copy(adst.at[dstb.at[j]], adb.at[j], gsem))
                    gathers.append(
                        pltpu.async_copy(ztab.at[srcb.at[j]], zb.at[j], gsem))
                for g in gathers:
                    g.wait()

                def wrow(j, carry2):
                    def wvec(v, carry3):
                        t = (asb[j, pl.ds(v * 16, 16)]
                             + adb[j, pl.ds(v * 16, 16)])
                        w16 = jnp.exp(_leaky(t, 0.2))
                        wb[j, pl.ds(v * 16, 16)] = w16
                        for l in range(16):
                            k = v * 16 + l
                            zb[j, k, :] = zb[j, k, :] * w16[l]
                        return carry3
                    return lax.fori_loop(0, CW // 16, wvec, carry2)
                lax.fori_loop(0, BLK, wrow, 0)

                scatters = []
                for j in range(BLK):
                    scatters.append(
                        pltpu.async_copy(zb.at[j], acc.at[dstb.at[j]],
                                         ssem, add=True))
                    if with_den:
                        scatters.append(
                            pltpu.async_copy(wb.at[j], dacc.at[dstb.at[j]],
                                             ssem, add=True))
                for h in scatters:
                    h.wait()
                return carry
            lax.fori_loop(0, NBLK, block_body, 0)

        @pl.when(c == 0)
        def _():
            run(zlo, True)

        @pl.when(c == 1)
        def _():
            run(zhi, False)

        plsc.subcore_barrier()

        @pl.when(c == 0)
        def _():
            pltpu.sync_copy(acc.at[pl.ds(s * RPT, RPT)],
                            nlo.at[pl.ds(s * RPT, RPT)])
            pltpu.sync_copy(dacc.at[pl.ds(s * RPT, RPT)],
                            dout.at[pl.ds(s * RPT, RPT)])

        @pl.when(c == 1)
        def _():
            pltpu.sync_copy(acc.at[pl.ds(s * RPT, RPT)],
                            nhi.at[pl.ds(s * RPT, RPT)])

    return edge_kernel


_edge = _make_edge_kernel()


def kernel(x, edge_index, batch, emb_W, emb_b,
           conv1_W, conv1_att_src, conv1_att_dst, conv1_b,
           conv2_W, conv2_att_src, conv2_att_dst, conv2_b,
           fc_W, fc_b, out_W, out_b):
    xp = jnp.pad(x, ((0, NPAD - N), (0, 0)))
    npd = EPAD - E
    fill = N + (jnp.arange(npd, dtype=jnp.int32) % (NPAD - N))
    ei = jnp.concatenate(
        [edge_index, jnp.stack([fill, fill])], axis=1).reshape(2, CHUNKS, CW)
    batch3 = jnp.pad(batch, (0, NPAD - N),
                     constant_values=-1).reshape(GRID, 1, BN)
    zrow = jnp.zeros((RPT, HALF), jnp.float32)
    zcol = jnp.zeros((RPT,), jnp.float32)

    att1 = jnp.stack([conv1_att_src, conv1_att_dst], axis=1)
    att2 = jnp.stack([conv2_att_src, conv2_att_dst], axis=1)

    zlo1, zhi1, as1, ad1 = _embed(
        xp, emb_W, emb_b.reshape(1, D_HID), conv1_W, att1)
    nlo1, nhi1, den1 = _edge(
        zlo1, zhi1, as1.reshape(NPAD), ad1.reshape(NPAD), ei, zrow, zcol)
    zlo2, zhi2, as2, ad2 = _mid(
        nlo1, nhi1, den1.reshape(NPAD, 1), as1, ad1, zlo1, zhi1,
        conv1_b.reshape(1, D_HID), conv2_W, att2)
    nlo2, nhi2, den2 = _edge(
        zlo2, zhi2, as2.reshape(NPAD), ad2.reshape(NPAD), ei, zrow, zcol)
    proba, logits = _final(
        nlo2, nhi2, den2.reshape(NPAD, 1), as2, ad2, zlo2, zhi2,
        conv2_b.reshape(1, D_HID), batch3, fc_W, fc_b.reshape(1, 64),
        out_W, out_b.reshape(1, 2))
    return (proba, logits)
